# Initial kernel scaffold; baseline (speedup 1.0000x reference)
import jax
import jax.numpy as jnp
from jax.experimental import pallas as pl


def kernel(dist, angle, idx_t, index_t, index_h):
    raise NotImplementedError("write your pallas kernel here")


# speedup vs baseline: 3.4234x; 3.4234x over previous
import functools

import jax
import jax.numpy as jnp
from jax import lax
from jax.experimental import pallas as pl
from jax.experimental.pallas import tpu as pltpu
from jax.experimental.pallas import tpu_sc as plsc

NCLS = 22


CHUNK = 256


def _sc_gather(dist, angle, qpair2d, pre_flat, h, w, L):
    info = plsc.get_sparse_core_info()
    NC, NS, LN = info.num_cores, info.num_subcores, info.num_lanes
    NW = NC * NS
    hb = h // NW
    hw_b = hb * w
    n_chunks = hb // CHUNK
    cw = CHUNK * w
    n_dma = 2 * CHUNK // 128
    mesh = plsc.VectorSubcoreMesh(core_axis_name="c", subcore_axis_name="s")

    @functools.partial(
        pl.kernel,
        out_type=(
            jax.ShapeDtypeStruct((h * w,), jnp.float32),
            jax.ShapeDtypeStruct((h * w,), jnp.float32),
            jax.ShapeDtypeStruct((NW, LN), jnp.float32),
            jax.ShapeDtypeStruct((NW, LN), jnp.float32),
        ),
        mesh=mesh,
        compiler_params=pltpu.CompilerParams(needs_layout_passes=False,
                                             use_tc_tiling_on_sc=False),
        scratch_types=(
            pltpu.VMEM((n_dma, 128), jnp.int32),
            pltpu.VMEM((2 * CHUNK, L), jnp.float32),
            pltpu.VMEM((2 * CHUNK, L), jnp.float32),
            pltpu.VMEM((cw,), jnp.int32),
            pltpu.VMEM((hw_b,), jnp.float32),
            pltpu.VMEM((hw_b,), jnp.float32),
            pltpu.VMEM((LN,), jnp.float32),
            pltpu.VMEM((LN,), jnp.float32),
            pltpu.SemaphoreType.DMA,
        ),
    )
    def k(dist_hbm, angle_hbm, qpair_hbm, pre_hbm,
          dist_g, angle_g, mins, maxs,
          idx_s, dist_rows, angle_rows, pre_v, dist_o, angle_o,
          min_v, max_v, sem):
        wid = lax.axis_index("s") * NC + lax.axis_index("c")
        inf = jnp.full((LN,), jnp.inf, dtype=jnp.float32)
        zero = jnp.zeros((LN,), dtype=jnp.float32)
        vmin, vmax = inf, -inf
        for ch in range(n_chunks):
            pltpu.sync_copy(
                qpair_hbm.at[pl.ds((wid * n_chunks + ch) * n_dma, n_dma)],
                idx_s)
            pltpu.sync_copy(
                pre_hbm.at[pl.ds(wid * hw_b + ch * cw, cw)], pre_v)
            cps = []
            for kk in range(n_dma):
                cps.append(pltpu.async_copy(
                    dist_hbm.at[idx_s.at[kk]],
                    dist_rows.at[pl.ds(kk * 128, 128)], sem))
                cps.append(pltpu.async_copy(
                    angle_hbm.at[idx_s.at[kk]],
                    angle_rows.at[pl.ds(kk * 128, 128)], sem))
            for cp in cps:
                cp.wait()

            def body(g, carry, _ch=ch):
                mn, mx = carry
                base = g * LN
                v = pre_v[pl.ds(base, LN)]
                col = v & 63
                row = (v >> 6) & 1023
                inv = v > 65535
                vd = plsc.load_gather(dist_rows, [row, col])
                va = plsc.load_gather(angle_rows, [row, col])
                vd = jnp.where(inv, zero, vd)
                va = jnp.where(inv, zero, va)
                dist_o[pl.ds(_ch * cw + base, LN)] = vd
                angle_o[pl.ds(_ch * cw + base, LN)] = va
                return jnp.minimum(mn, vd), jnp.maximum(mx, vd)

            vmin, vmax = lax.fori_loop(0, cw // LN, body, (vmin, vmax))

        min_v[...] = vmin
        max_v[...] = vmax
        pltpu.sync_copy(dist_o, dist_g.at[pl.ds(wid * hw_b, hw_b)])
        pltpu.sync_copy(angle_o, angle_g.at[pl.ds(wid * hw_b, hw_b)])
        pltpu.sync_copy(min_v, mins.at[wid])
        pltpu.sync_copy(max_v, maxs.at[wid])

    return k(dist, angle, qpair2d, pre_flat)


def _tc_assemble(idx_t, dist_g, angle_g, mins, maxs, h, w, interpret=False):
    out_w = NCLS * w + 2 * w
    BH = 512
    grid = (h // BH,)

    def body(idx_ref, dist_ref, angle_ref, mins_ref, maxs_ref, out_ref):
        gmin = jnp.min(mins_ref[...])
        gmax = jnp.max(maxs_ref[...])
        scale = 1.0 / (gmax - gmin)
        idx = idx_ref[...]
        cls_iota = lax.broadcasted_iota(jnp.int32, (BH, NCLS), 1)
        for j in range(w):
            oh = (idx[:, j:j + 1] == cls_iota).astype(jnp.float32)
            out_ref[:, NCLS * j:NCLS * (j + 1)] = oh
        out_ref[:, NCLS * w:NCLS * w + w] = (dist_ref[...] - gmin) * scale
        out_ref[:, NCLS * w + w:] = angle_ref[...]

    return pl.pallas_call(
        body,
        grid=grid,
        in_specs=[
            pl.BlockSpec((BH, w), lambda i: (i, 0)),
            pl.BlockSpec((BH, w), lambda i: (i, 0)),
            pl.BlockSpec((BH, w), lambda i: (i, 0)),
            pl.BlockSpec(mins.shape, lambda i: (0, 0)),
            pl.BlockSpec(maxs.shape, lambda i: (0, 0)),
        ],
        out_specs=pl.BlockSpec((BH, out_w), lambda i: (i, 0)),
        out_shape=jax.ShapeDtypeStruct((h, out_w), jnp.float32),
        interpret=interpret,
    )(idx_t, dist_g, angle_g, mins, maxs)


def _precompute(index_h, index_t, h, w, L):
    pitch = ((L + 7) // 8) * 8
    r = index_h.astype(jnp.int32)
    w0 = r * pitch
    q = w0 // L
    ob = w0 - q * L
    o = ob[:, None] + index_t
    inv = (index_t >= L).astype(jnp.int32)
    p = (jnp.arange(h, dtype=jnp.int32) % CHUNK)[:, None]
    a = (pitch * 128) * (p // 64) + 2 * L * (p % 64) + o
    rowloc = a // pitch
    col = a - pitch * rowloc
    pre = (col | (rowloc << 6) | (inv << 16)).reshape(-1)
    qpair = jnp.stack([q, q + 1], axis=-1).reshape(-1, 128)
    return qpair, pre


def kernel(dist, angle, idx_t, index_t, index_h):
    N, L = dist.shape
    h, w = idx_t.shape
    qpair, pre = _precompute(index_h, index_t, h, w, L)
    dist_g, angle_g, mins, maxs = _sc_gather(dist, angle, qpair, pre,
                                             h, w, L)
    return _tc_assemble(idx_t, dist_g.reshape(h, w), angle_g.reshape(h, w),
                        mins, maxs, h, w)


# speedup vs baseline: 4.1860x; 1.2228x over previous
import functools

import jax
import jax.numpy as jnp
from jax import lax
from jax.experimental import pallas as pl
from jax.experimental.pallas import tpu as pltpu
from jax.experimental.pallas import tpu_sc as plsc

NCLS = 22


CHUNK = 256


def _sc_gather(dist, angle, qpair2d, pre_flat, h, w, L):
    info = plsc.get_sparse_core_info()
    NC, NS, LN = info.num_cores, info.num_subcores, info.num_lanes
    NW = NC * NS
    hb = h // NW
    hw_b = hb * w
    n_chunks = hb // CHUNK
    cw = CHUNK * w
    n_dma = 2 * CHUNK // 128
    mesh = plsc.VectorSubcoreMesh(core_axis_name="c", subcore_axis_name="s")

    @functools.partial(
        pl.kernel,
        out_type=(
            jax.ShapeDtypeStruct((h * w,), jnp.float32),
            jax.ShapeDtypeStruct((h * w,), jnp.float32),
            jax.ShapeDtypeStruct((NW, LN), jnp.float32),
            jax.ShapeDtypeStruct((NW, LN), jnp.float32),
        ),
        mesh=mesh,
        compiler_params=pltpu.CompilerParams(needs_layout_passes=False,
                                             use_tc_tiling_on_sc=False),
        scratch_types=(
            pltpu.VMEM((n_dma, 128), jnp.int32),
            pltpu.VMEM((2 * CHUNK, L), jnp.float32),
            pltpu.VMEM((2 * CHUNK, L), jnp.float32),
            pltpu.VMEM((cw,), jnp.int32),
            pltpu.VMEM((hw_b,), jnp.float32),
            pltpu.VMEM((hw_b,), jnp.float32),
            pltpu.VMEM((LN,), jnp.float32),
            pltpu.VMEM((LN,), jnp.float32),
            pltpu.SemaphoreType.DMA,
        ),
    )
    def k(dist_hbm, angle_hbm, qpair_hbm, pre_hbm,
          dist_g, angle_g, mins, maxs,
          idx_s, dist_rows, angle_rows, pre_v, dist_o, angle_o,
          min_v, max_v, sem):
        wid = lax.axis_index("s") * NC + lax.axis_index("c")
        inf = jnp.full((LN,), jnp.inf, dtype=jnp.float32)
        zero = jnp.zeros((LN,), dtype=jnp.float32)
        vmin, vmax = inf, -inf
        for ch in range(n_chunks):
            pltpu.sync_copy(
                qpair_hbm.at[pl.ds((wid * n_chunks + ch) * n_dma, n_dma)],
                idx_s)
            pltpu.sync_copy(
                pre_hbm.at[pl.ds(wid * hw_b + ch * cw, cw)], pre_v)
            cps = []
            for kk in range(n_dma):
                cps.append(pltpu.async_copy(
                    dist_hbm.at[idx_s.at[kk]],
                    dist_rows.at[pl.ds(kk * 128, 128)], sem))
                cps.append(pltpu.async_copy(
                    angle_hbm.at[idx_s.at[kk]],
                    angle_rows.at[pl.ds(kk * 128, 128)], sem))
            for cp in cps:
                cp.wait()

            def body(g, carry, _ch=ch):
                mn, mx = carry
                base = g * LN
                v = pre_v[pl.ds(base, LN)]
                col = v & 63
                row = (v >> 6) & 1023
                inv = v > 65535
                vd = plsc.load_gather(dist_rows, [row, col])
                va = plsc.load_gather(angle_rows, [row, col])
                vd = jnp.where(inv, zero, vd)
                va = jnp.where(inv, zero, va)
                dist_o[pl.ds(_ch * cw + base, LN)] = vd
                angle_o[pl.ds(_ch * cw + base, LN)] = va
                return jnp.minimum(mn, vd), jnp.maximum(mx, vd)

            vmin, vmax = lax.fori_loop(0, cw // LN, body, (vmin, vmax))

        min_v[...] = vmin
        max_v[...] = vmax
        pltpu.sync_copy(dist_o, dist_g.at[pl.ds(wid * hw_b, hw_b)])
        pltpu.sync_copy(angle_o, angle_g.at[pl.ds(wid * hw_b, hw_b)])
        pltpu.sync_copy(min_v, mins.at[wid])
        pltpu.sync_copy(max_v, maxs.at[wid])

    return k(dist, angle, qpair2d, pre_flat)


def _tc_assemble(idx_t, dist_g, angle_g, mins, maxs, h, w, interpret=False):
    out_w = NCLS * w + 2 * w
    BH = 2048
    grid = (h // BH,)

    def body(idx_ref, dist_ref, angle_ref, mins_ref, maxs_ref, out_ref):
        gmin = jnp.min(mins_ref[...])
        gmax = jnp.max(maxs_ref[...])
        scale = 1.0 / (gmax - gmin)
        idxf = idx_ref[...].astype(jnp.bfloat16)
        qj = lax.broadcasted_iota(jnp.int32, (w, NCLS * w), 1) // NCLS
        jj = lax.broadcasted_iota(jnp.int32, (w, NCLS * w), 0)
        sel = (qj == jj).astype(jnp.bfloat16)
        rep = jnp.dot(idxf, sel, preferred_element_type=jnp.float32)
        cls = (lax.broadcasted_iota(jnp.int32, (BH, NCLS * w), 1)
               % NCLS).astype(jnp.float32)
        out_ref[:, :NCLS * w] = (rep == cls).astype(jnp.float32)
        out_ref[:, NCLS * w:NCLS * w + w] = (dist_ref[...] - gmin) * scale
        out_ref[:, NCLS * w + w:] = angle_ref[...]

    return pl.pallas_call(
        body,
        grid=grid,
        in_specs=[
            pl.BlockSpec((BH, w), lambda i: (i, 0)),
            pl.BlockSpec((BH, w), lambda i: (i, 0)),
            pl.BlockSpec((BH, w), lambda i: (i, 0)),
            pl.BlockSpec(mins.shape, lambda i: (0, 0)),
            pl.BlockSpec(maxs.shape, lambda i: (0, 0)),
        ],
        out_specs=pl.BlockSpec((BH, out_w), lambda i: (i, 0)),
        out_shape=jax.ShapeDtypeStruct((h, out_w), jnp.float32),
        interpret=interpret,
    )(idx_t, dist_g, angle_g, mins, maxs)


def _precompute(index_h, index_t, h, w, L):
    pitch = ((L + 7) // 8) * 8
    r = index_h.astype(jnp.int32)
    w0 = r * pitch
    q = w0 // L
    ob = w0 - q * L
    o = ob[:, None] + index_t
    inv = (index_t >= L).astype(jnp.int32)
    p = (jnp.arange(h, dtype=jnp.int32) % CHUNK)[:, None]
    a = (pitch * 128) * (p // 64) + 2 * L * (p % 64) + o
    rowloc = a // pitch
    col = a - pitch * rowloc
    pre = (col | (rowloc << 6) | (inv << 16)).reshape(-1)
    qpair = jnp.stack([q, q + 1], axis=-1).reshape(-1, 128)
    return qpair, pre


def kernel(dist, angle, idx_t, index_t, index_h):
    N, L = dist.shape
    h, w = idx_t.shape
    qpair, pre = _precompute(index_h, index_t, h, w, L)
    dist_g, angle_g, mins, maxs = _sc_gather(dist, angle, qpair, pre,
                                             h, w, L)
    return _tc_assemble(idx_t, dist_g.reshape(h, w), angle_g.reshape(h, w),
                        mins, maxs, h, w)
